# baseline (speedup 1.0000x reference)
import functools

import jax
import jax.numpy as jnp
from jax import lax
from jax.experimental import pallas as pl
from jax.experimental.pallas import tpu as pltpu
from jax.experimental.pallas import tpu_sc as plsc

_B = 16384
_D = 32
_P = 8
_L = 16
_NW = 32
_ROWS = _B // _NW


def _sc_kernel(q_hbm, proj_hbm, w_hbm, b_hbm, out_hbm, q_v, o_v, proj_v, w_v, b_v):
    wid = lax.axis_index("s") * 2 + lax.axis_index("c")
    base = wid * _ROWS

    pltpu.sync_copy(q_hbm.at[pl.ds(base, _ROWS)], q_v)
    pltpu.sync_copy(proj_hbm, proj_v)
    pltpu.sync_copy(w_hbm.at[0], w_v.at[pl.ds(0, _P)])
    pltpu.sync_copy(b_hbm.at[0], b_v)

    w_vec = w_v[...]
    w_s = [w_vec[p] for p in range(_P)]
    m_chunks = []
    for k in range(_D):
        row = []
        for c in range(2):
            acc = w_s[0] * proj_v[0, k, pl.ds(c * _L, _L)]
            for p in range(1, _P):
                acc = acc + w_s[p] * proj_v[p, k, pl.ds(c * _L, _L)]
            row.append(acc)
        m_chunks.append(row)

    bias0 = b_v[pl.ds(0, _L)]
    bias1 = b_v[pl.ds(_L, _L)]

    def body(i, carry):
        acc0 = bias0
        acc1 = bias1
        q0 = q_v[i, pl.ds(0, _L)]
        q1 = q_v[i, pl.ds(_L, _L)]
        for k in range(_D):
            s = q0[k] if k < _L else q1[k - _L]
            acc0 = acc0 + s * m_chunks[k][0]
            acc1 = acc1 + s * m_chunks[k][1]
        o_v[i, pl.ds(0, _L)] = acc0
        o_v[i, pl.ds(_L, _L)] = acc1
        return carry

    lax.fori_loop(0, _ROWS, body, 0)

    pltpu.sync_copy(o_v, out_hbm.at[pl.ds(base, _ROWS)])


def kernel(query_embedding, projector, W, b):
    mesh = plsc.VectorSubcoreMesh(core_axis_name="c", subcore_axis_name="s")
    k = functools.partial(
        pl.kernel,
        mesh=mesh,
        compiler_params=pltpu.CompilerParams(use_tc_tiling_on_sc=False),
        out_type=jax.ShapeDtypeStruct((_B, _D), jnp.float32),
        scratch_types=[
            pltpu.VMEM((_ROWS, _D), jnp.float32),
            pltpu.VMEM((_ROWS, _D), jnp.float32),
            pltpu.VMEM((_P, _D, _D), jnp.float32),
            pltpu.VMEM((_L,), jnp.float32),
            pltpu.VMEM((_D,), jnp.float32),
        ],
    )(_sc_kernel)
    out = k(query_embedding, projector, W, b)
    return out[:, None, :]
